# Initial kernel scaffold; baseline (speedup 1.0000x reference)
import jax
import jax.numpy as jnp
from jax.experimental import pallas as pl


def kernel(x, edge_index, edge_attr, We, be, W0l, b0, W0r, W1l, b1, W1r, W2l, b2, W2r, Wb, bb, Wrt, brt):
    raise NotImplementedError("write your pallas kernel here")


# speedup vs baseline: 3.2677x; 3.2677x over previous
import functools

import jax
import jax.numpy as jnp
from jax import lax
from jax.experimental import pallas as pl
from jax.experimental.pallas import tpu as pltpu
from jax.experimental.pallas import tpu_sc as plsc

_NC = 2
_NS = 16
_NW = _NC * _NS
_C = 128
_F32 = jnp.float32


def _mesh():
    return plsc.VectorSubcoreMesh(core_axis_name="c", subcore_axis_name="s")


def _row_split(n):
    span = -(-(n // _NS) // 8) * 8
    tail = n - span * (_NS - 1)
    assert tail > 0 and tail % 8 == 0 and span % 8 == 0
    return span, tail


def _for_my_rows(sid, span, tail, do):
    off = pl.multiple_of(sid * span, 8)

    @pl.when(sid < _NS - 1)
    def _():
        do(off, span)

    @pl.when(sid == _NS - 1)
    def _():
        do(off, tail)


def _zero_rows(z_v, acc, off, sz):
    for j in range(sz // 128):
        pltpu.sync_copy(z_v, acc.at[pl.ds(off + j * 128, 128)])
    rem = sz % 128
    if rem:
        pltpu.sync_copy(z_v.at[pl.ds(0, rem)],
                        acc.at[pl.ds(off + (sz // 128) * 128, rem)])


def _phase_a(edge_attr, srcs, dsts, n):
    e = srcs.shape[0]
    nch = e // _C
    kmax = (nch + _NS - 1) // _NS
    span, tail = _row_split(n)

    @functools.partial(
        pl.kernel,
        out_type=jax.ShapeDtypeStruct((_NC, n, 16), _F32),
        mesh=_mesh(),
        scratch_types=[
            pltpu.VMEM((_C, 16), _F32),
            pltpu.VMEM((_C,), jnp.int32),
            pltpu.VMEM((_C,), jnp.int32),
            pltpu.VMEM((_C, 16), _F32),
            pltpu.VMEM((_C, 16), _F32),
            pltpu.VMEM((128, 16), _F32),
            pltpu.VMEM_SHARED((n, 16), _F32),
        ],
        compiler_params=pltpu.CompilerParams(use_tc_tiling_on_sc=False),
    )
    def body(ea_hbm, src_hbm, dst_hbm, out,
             attr_v, src_v, dst_v, e01_v, e0_v, z_v, acc):
        cid = lax.axis_index("c")
        sid = lax.axis_index("s")

        lane = lax.iota(jnp.int32, 16)
        e01 = jnp.where(lane < 2, 1.0, 0.0).astype(_F32)
        e0 = jnp.where(lane < 1, 1.0, 0.0).astype(_F32)
        zero16 = jnp.zeros((16,), _F32)

        def init_const(i, _):
            e01_v[i, :] = e01
            e0_v[i, :] = e0
            return 0
        lax.fori_loop(0, _C, init_const, 0)

        def init_zero(i, _):
            z_v[i, :] = zero16
            return 0
        lax.fori_loop(0, 128, init_zero, 0)

        _for_my_rows(sid, span, tail,
                     lambda off, sz: _zero_rows(z_v, acc, off, sz))
        plsc.subcore_barrier()

        def chunk(k, _):
            m = k * _NS + sid

            @pl.when(m < nch)
            def _():
                base = m * _C
                pltpu.sync_copy(src_hbm.at[pl.ds(base, _C)], src_v)
                pltpu.sync_copy(dst_hbm.at[pl.ds(base, _C)], dst_v)

                @pl.when(cid == 0)
                def _():
                    pltpu.sync_copy(ea_hbm.at[pl.ds(base, _C)], attr_v)
                    pltpu.sync_copy(attr_v, acc.at[dst_v], add=True)
                    pltpu.sync_copy(attr_v, acc.at[src_v], add=True)

                @pl.when(cid == 1)
                def _():
                    pltpu.sync_copy(e01_v, acc.at[dst_v], add=True)
                    pltpu.sync_copy(e0_v, acc.at[src_v], add=True)
            return 0
        lax.fori_loop(0, kmax, chunk, 0)
        plsc.subcore_barrier()

        _for_my_rows(sid, span, tail, lambda off, sz: pltpu.sync_copy(
            acc.at[pl.ds(off, sz)], out.at[cid, pl.ds(off, sz)]))

    return body(edge_attr, srcs, dsts)


def _phase_b(t00, t01, t10, t11, srcs, dsts, n):
    e = srcs.shape[0]
    nch = e // _C
    kmax = (nch + _NS - 1) // _NS
    span, tail = _row_split(n)

    @functools.partial(
        pl.kernel,
        out_type=jax.ShapeDtypeStruct((2, _NC, n, 16), _F32),
        mesh=_mesh(),
        scratch_types=[
            pltpu.VMEM((_C,), jnp.int32),
            pltpu.VMEM((_C,), jnp.int32),
            pltpu.VMEM((_C, 16), _F32),
            pltpu.VMEM((128, 16), _F32),
            pltpu.VMEM_SHARED((n, 16), _F32),
        ],
        compiler_params=pltpu.CompilerParams(use_tc_tiling_on_sc=False),
    )
    def body(t00_hbm, t01_hbm, t10_hbm, t11_hbm, src_hbm, dst_hbm, part_out,
             src_v, dst_v, rows_v, z_v, acc):
        cid = lax.axis_index("c")
        sid = lax.axis_index("s")

        zero16 = jnp.zeros((16,), _F32)

        def init_zero(i, _):
            z_v[i, :] = zero16
            return 0
        lax.fori_loop(0, 128, init_zero, 0)

        _for_my_rows(sid, span, tail,
                     lambda off, sz: _zero_rows(z_v, acc, off, sz))
        plsc.subcore_barrier()

        for p, (ta, tb) in enumerate(((t00_hbm, t10_hbm), (t01_hbm, t11_hbm))):
            def chunk(k, _, ta=ta, tb=tb):
                m = k * _NS + sid

                @pl.when(m < nch)
                def _():
                    base = m * _C
                    pltpu.sync_copy(src_hbm.at[pl.ds(base, _C)], src_v)
                    pltpu.sync_copy(dst_hbm.at[pl.ds(base, _C)], dst_v)

                    @pl.when(cid == 0)
                    def _():
                        pltpu.sync_copy(ta.at[src_v], rows_v)

                    @pl.when(cid == 1)
                    def _():
                        pltpu.sync_copy(tb.at[src_v], rows_v)

                    pltpu.sync_copy(rows_v, acc.at[dst_v], add=True)
                return 0
            lax.fori_loop(0, kmax, chunk, 0)
            plsc.subcore_barrier()

            def flush(off, sz, p=p):
                pltpu.sync_copy(acc.at[pl.ds(off, sz)],
                                part_out.at[p, cid, pl.ds(off, sz)])
                if p == 0:
                    _zero_rows(z_v, acc, off, sz)
            _for_my_rows(sid, span, tail, flush)
            if p == 0:
                plsc.subcore_barrier()

    return body(t00, t01, t10, t11, srcs, dsts)


def _phase_c(tbl, srcs, dsts, n):
    e = srcs.shape[0]
    nch = e // _C
    kmax = (nch + _NW - 1) // _NW
    span, tail = _row_split(n)

    @functools.partial(
        pl.kernel,
        out_type=jax.ShapeDtypeStruct((_NC, n, 16), _F32),
        mesh=_mesh(),
        scratch_types=[
            pltpu.VMEM((_C,), jnp.int32),
            pltpu.VMEM((_C,), jnp.int32),
            pltpu.VMEM((_C, 16), _F32),
            pltpu.VMEM((128, 16), _F32),
            pltpu.VMEM_SHARED((n, 16), _F32),
        ],
        compiler_params=pltpu.CompilerParams(use_tc_tiling_on_sc=False),
    )
    def body(tbl_hbm, src_hbm, dst_hbm, part_out,
             src_v, dst_v, rows_v, z_v, acc):
        cid = lax.axis_index("c")
        sid = lax.axis_index("s")
        w = cid * _NS + sid

        zero16 = jnp.zeros((16,), _F32)

        def init_zero(i, _):
            z_v[i, :] = zero16
            return 0
        lax.fori_loop(0, 128, init_zero, 0)

        _for_my_rows(sid, span, tail,
                     lambda off, sz: _zero_rows(z_v, acc, off, sz))
        plsc.subcore_barrier()

        def chunk(k, _):
            m = k * _NW + w

            @pl.when(m < nch)
            def _():
                base = m * _C
                pltpu.sync_copy(src_hbm.at[pl.ds(base, _C)], src_v)
                pltpu.sync_copy(dst_hbm.at[pl.ds(base, _C)], dst_v)
                pltpu.sync_copy(tbl_hbm.at[src_v], rows_v)
                pltpu.sync_copy(rows_v, acc.at[dst_v], add=True)
            return 0
        lax.fori_loop(0, kmax, chunk, 0)
        plsc.subcore_barrier()

        _for_my_rows(sid, span, tail, lambda off, sz: pltpu.sync_copy(
            acc.at[pl.ds(off, sz)], part_out.at[cid, pl.ds(off, sz)]))

    return body(tbl, srcs, dsts)


_BN = 2000


def _full(shape):
    return pl.BlockSpec(shape, lambda i: (0,) * len(shape))


def _rows(shape):
    return pl.BlockSpec(shape, lambda i: (i,) + (0,) * (len(shape) - 1))


def _mid(shape):
    return pl.BlockSpec(shape, lambda i: (0, i) + (0,) * (len(shape) - 2))


def _split4(hl):
    return hl[:, :16], hl[:, 16:32], hl[:, 32:48], hl[:, 48:]


_T4_SPECS = [_rows((_BN, 16))] * 4


def _t4_shapes(n):
    return [jax.ShapeDtypeStruct((n, 16), _F32)] * 4


def _tc_stage_a(x, s, d, We, be, W0la, W0lb, W0ra, W0rb, b0, n):
    def body(x_r, s_r, d_r, we_r, be_r, wla_r, wlb_r, wra_r, wrb_r, b0_r,
             t00_r, t01_r, t10_r, t11_r, hr_r):
        cnt = d_r[:, 0:1]
        agg = (jnp.dot(s_r[...], we_r[...], preferred_element_type=_F32)
               + cnt * be_r[...]) / jnp.maximum(cnt, 1.0)
        xv = x_r[...]
        hl = (jnp.dot(xv, wla_r[...], preferred_element_type=_F32)
              + jnp.dot(agg, wlb_r[...], preferred_element_type=_F32))
        t00_r[...], t01_r[...], t10_r[...], t11_r[...] = _split4(hl)
        hr_r[...] = (jnp.dot(xv, wra_r[...], preferred_element_type=_F32)
                     + jnp.dot(agg, wrb_r[...], preferred_element_type=_F32)
                     + b0_r[...])

    return pl.pallas_call(
        body,
        grid=(n // _BN,),
        in_specs=[
            _rows((_BN, 128)), _rows((_BN, 16)), _rows((_BN, 16)),
            _full((16, 16)), _full((1, 16)),
            _full((128, 64)), _full((16, 64)),
            _full((128, 64)), _full((16, 64)), _full((1, 64)),
        ],
        out_specs=_T4_SPECS + [_rows((_BN, 64))],
        out_shape=_t4_shapes(n) + [jax.ShapeDtypeStruct((n, 64), _F32)],
    )(x, s, d, We, be, W0la, W0lb, W0ra, W0rb, b0)


def _elu(pre):
    return jnp.where(pre > 0, pre, jnp.exp(jnp.minimum(pre, 0.0)) - 1.0)


def _p4spec():
    return pl.BlockSpec((2, _NC, _BN, 16), lambda i: (0, 0, i, 0))


def _pcat(p_r):
    return jnp.concatenate([p_r[0, 0], p_r[1, 0], p_r[0, 1], p_r[1, 1]],
                           axis=-1)


def _tc_stage_mid(p, d, hr_prev, Wl, b, Wr, n):
    def body(p_r, d_r, hrp_r, wl_r, b_r, wr_r,
             t00_r, t01_r, t10_r, t11_r, hr_r):
        ps = _pcat(p_r)
        indeg = d_r[:, 1:2]
        h = _elu(ps / jnp.maximum(indeg, 1.0) + hrp_r[...])
        hl = jnp.dot(h, wl_r[...], preferred_element_type=_F32)
        t00_r[...], t01_r[...], t10_r[...], t11_r[...] = _split4(hl)
        hr_r[...] = jnp.dot(h, wr_r[...], preferred_element_type=_F32) + b_r[...]

    return pl.pallas_call(
        body,
        grid=(n // _BN,),
        in_specs=[
            _p4spec(), _rows((_BN, 16)), _rows((_BN, 64)),
            _full((64, 64)), _full((1, 64)), _full((64, 64)),
        ],
        out_specs=_T4_SPECS + [_rows((_BN, 64))],
        out_shape=_t4_shapes(n) + [jax.ShapeDtypeStruct((n, 64), _F32)],
    )(p, d, hr_prev, Wl, b, Wr)


def _tc_stage_fold(p, d, hr_prev, W2l, b2, W2r, Whead, bhead, n):
    def body(p_r, d_r, hrp_r, wl_r, b_r, wr_r, wh_r, bh_r, tbl_r, base_r):
        ps = _pcat(p_r)
        indeg = d_r[:, 1:2]
        h = _elu(ps / jnp.maximum(indeg, 1.0) + hrp_r[...])
        wlh = jnp.dot(wl_r[...], wh_r[...], preferred_element_type=_F32)
        wrh = jnp.dot(wr_r[...], wh_r[...], preferred_element_type=_F32)
        t = jnp.dot(h, wlh, preferred_element_type=_F32)
        tbl_r[...] = jnp.concatenate(
            [t, jnp.zeros((t.shape[0], 14), _F32)], axis=-1)
        base_r[...] = (jnp.dot(h, wrh, preferred_element_type=_F32)
                       + jnp.dot(b_r[...], wh_r[...],
                                 preferred_element_type=_F32)
                       + bh_r[...])

    return pl.pallas_call(
        body,
        grid=(n // _BN,),
        in_specs=[
            _p4spec(), _rows((_BN, 16)), _rows((_BN, 64)),
            _full((64, 64)), _full((1, 64)), _full((64, 64)),
            _full((64, 2)), _full((1, 2)),
        ],
        out_specs=[_rows((_BN, 16)), _rows((_BN, 2))],
        out_shape=[
            jax.ShapeDtypeStruct((n, 16), _F32),
            jax.ShapeDtypeStruct((n, 2), _F32),
        ],
    )(p, d, hr_prev, W2l, b2, W2r, Whead, bhead)


def _tc_stage_final(p, d, base, n):
    def body(p_r, d_r, base_r, out_r):
        ps = p_r[0] + p_r[1]
        indeg = d_r[:, 1:2]
        out_r[...] = ps[:, :2] / jnp.maximum(indeg, 1.0) + base_r[...]

    return pl.pallas_call(
        body,
        grid=(n // _BN,),
        in_specs=[_mid((_NC, _BN, 16)), _rows((_BN, 16)), _rows((_BN, 2))],
        out_specs=_rows((_BN, 2)),
        out_shape=jax.ShapeDtypeStruct((n, 2), _F32),
    )(p, d, base)


def kernel(x, edge_index, edge_attr, We, be, W0l, b0, W0r, W1l, b1, W1r,
           W2l, b2, W2r, Wb, bb, Wrt, brt):
    n = x.shape[0]
    srcs = edge_index[0]
    dsts = edge_index[1]

    sd = _phase_a(edge_attr, srcs, dsts, n)
    s, d = sd[0], sd[1]

    t00, t01, t10, t11, hr = _tc_stage_a(
        x, s, d, We, be.reshape(1, 16),
        W0l[:128], W0l[128:], W0r[:128], W0r[128:], b0.reshape(1, 64), n)

    p = _phase_b(t00, t01, t10, t11, srcs, dsts, n)
    t00, t01, t10, t11, hr = _tc_stage_mid(
        p, d, hr, W1l, b1.reshape(1, 64), W1r, n)

    p = _phase_b(t00, t01, t10, t11, srcs, dsts, n)
    Whead = jnp.concatenate([Wb, Wrt], axis=1)
    bhead = jnp.concatenate([bb, brt]).reshape(1, 2)
    tbl3, bout = _tc_stage_fold(
        p, d, hr, W2l, b2.reshape(1, 64), W2r, Whead, bhead, n)

    p = _phase_c(tbl3, srcs, dsts, n)
    return _tc_stage_final(p, d, bout, n)
